# baseline (speedup 1.0000x reference)
import functools

import jax
import jax.numpy as jnp
from jax import lax
from jax.experimental import pallas as pl
from jax.experimental.pallas import tpu as pltpu
from jax.experimental.pallas import tpu_sc as plsc

_B, _C, _H, _W = 16, 256, 32, 32
_HW = _H * _W
_T = _B * _HW
_K = 8192
_NBLK = 2048
_NJ = _K // _NBLK
_BETA = 0.25

def _dist_body(x_ref, w_ref, w2_ref, idx_ref, loss_ref, rmin_ref, acc_ref,
               mm_ref):
    b = pl.program_id(0)
    j = pl.program_id(1)
    x = x_ref[0]
    wblk = w_ref[...]
    mm = lax.dot_general(wblk, x, (((1,), (0,)), ((), ())),
                         preferred_element_type=jnp.float32)
    mm_ref[...] = mm
    inf = jnp.float32(3.4028235e38)
    base_idx = lax.broadcasted_iota(jnp.int32, (8, _HW), 0).astype(jnp.float32)

    def _chunk(r, carry):
        vmin8, varg8 = carry
        dc = mm_ref[pl.ds(r * 8, 8), :] + w2_ref[pl.ds(r * 8, 8), :]
        idxc = base_idx + (8.0 * r.astype(jnp.float32))
        better = dc < vmin8
        return (jnp.where(better, dc, vmin8), jnp.where(better, idxc, varg8))

    vmin8, varg8 = lax.fori_loop(
        0, _NBLK // 8, _chunk,
        (jnp.full((8, _HW), inf, jnp.float32), jnp.zeros((8, _HW), jnp.float32)),
        unroll=4)
    bmin = jnp.min(vmin8, axis=0)
    cand = jnp.where(vmin8 == bmin[None, :], varg8, inf)
    barg = jnp.min(cand, axis=0).astype(jnp.int32) + j * _NBLK

    @pl.when(j == 0)
    def _init():
        rmin_ref[...] = bmin
        idx_ref[0, 0, :] = barg

    @pl.when(j > 0)
    def _update():
        old = rmin_ref[...]
        better = bmin < old
        rmin_ref[...] = jnp.where(better, bmin, old)
        idx_ref[0, 0, :] = jnp.where(better, barg, idx_ref[0, 0, :])

    @pl.when(j == _NJ - 1)
    def _loss():
        z2 = jnp.sum(x * x, axis=0)
        part = jnp.sum(z2 + rmin_ref[...])
        acc = jnp.where(b == 0, 0.0, acc_ref[0])
        acc_ref[0] = acc + part

        @pl.when(b == _B - 1)
        def _write():
            loss_ref[...] = jnp.full((1, 128), acc_ref[0] * ((1.0 + _BETA) / float(_T * _C)), jnp.float32)


_dist = pl.pallas_call(
    _dist_body,
    grid=(_B, _NJ),
    in_specs=[
        pl.BlockSpec((1, _C, _HW), lambda b, j: (b, 0, 0)),
        pl.BlockSpec((_NBLK, _C), lambda b, j: (j, 0)),
        pl.BlockSpec((_NBLK, 1), lambda b, j: (j, 0)),
    ],
    out_specs=[
        pl.BlockSpec((1, 1, _HW), lambda b, j: (b, 0, 0)),
        pl.BlockSpec((1, 128), lambda b, j: (0, 0)),
    ],
    out_shape=[
        jax.ShapeDtypeStruct((_B, 1, _HW), jnp.int32),
        jax.ShapeDtypeStruct((1, 128), jnp.float32),
    ],
    scratch_shapes=[
        pltpu.VMEM((_HW,), jnp.float32),
        pltpu.SMEM((1,), jnp.float32),
        pltpu.VMEM((_NBLK, _HW), jnp.float32),
    ],
)

_NC, _NS = 2, 16
_NW = _NC * _NS
_BPW = _T // _NW
_CH = 128
_NCH = _BPW // _CH


def _gather_body(tab_ref, idx_ref, out_ref, idx_v, rows_v, sem):
    wid = lax.axis_index("s") * _NC + lax.axis_index("c")
    pltpu.sync_copy(idx_ref.at[pl.ds(wid * _NCH, _NCH)], idx_v)
    for c in range(_NCH):
        pltpu.async_copy(tab_ref.at[idx_v.at[c]], rows_v, sem).wait()
        pltpu.sync_copy(rows_v, out_ref.at[pl.ds(wid * _BPW + c * _CH, _CH)])


@functools.lru_cache(maxsize=1)
def _make_gather():
    return functools.partial(
        pl.kernel,
        out_type=jax.ShapeDtypeStruct((_T, _C), jnp.float32),
        mesh=plsc.VectorSubcoreMesh(core_axis_name="c", subcore_axis_name="s"),
        scratch_types=[
            pltpu.VMEM((_NCH, _CH), jnp.int32),
            pltpu.VMEM((_CH, _C), jnp.float32),
            pltpu.SemaphoreType.DMA,
        ],
    )(_gather_body)


def kernel(z, embedding_weight):
    z_r = z.reshape(_B, _C, _HW)
    w2 = jnp.sum(embedding_weight ** 2, axis=1)
    idx3, loss2 = _dist(z_r, -2.0 * embedding_weight, w2.reshape(_K, 1))
    zq2 = _make_gather()(embedding_weight, idx3.reshape(_NW * _NCH, _CH))
    z_q_out = jnp.moveaxis(zq2.reshape(_B, _H, _W, _C), -1, 1)
    return z_q_out, loss2[0, 0], idx3.reshape(_B, _H, _W)


# speedup vs baseline: 2.8093x; 2.8093x over previous
import functools

import jax
import jax.numpy as jnp
from jax import lax
from jax.experimental import pallas as pl
from jax.experimental.pallas import tpu as pltpu
from jax.experimental.pallas import tpu_sc as plsc

_B, _C, _H, _W = 16, 256, 32, 32
_HW = _H * _W
_T = _B * _HW
_K = 8192
_NBLK = 2048
_NJ = _K // _NBLK
_BETA = 0.25

def _dist_body(x_ref, w_ref, w2_ref, idx_ref, loss_ref, rmin_ref, acc_ref):
    b = pl.program_id(0)
    j = pl.program_id(1)
    x = x_ref[0]
    wblk = w_ref[...]
    mm = lax.dot_general(wblk, x, (((1,), (0,)), ((), ())),
                         preferred_element_type=jnp.float32)
    d = w2_ref[...] + mm
    bmin = jnp.min(d, axis=0)
    barg = jnp.argmin(d, axis=0).astype(jnp.int32) + j * _NBLK

    @pl.when(j == 0)
    def _init():
        rmin_ref[...] = bmin
        idx_ref[0, 0, :] = barg

    @pl.when(j > 0)
    def _update():
        old = rmin_ref[...]
        better = bmin < old
        rmin_ref[...] = jnp.where(better, bmin, old)
        idx_ref[0, 0, :] = jnp.where(better, barg, idx_ref[0, 0, :])

    @pl.when(j == _NJ - 1)
    def _loss():
        z2 = jnp.sum(x * x, axis=0)
        part = jnp.sum(z2 + rmin_ref[...])
        acc = jnp.where(b == 0, 0.0, acc_ref[0])
        acc_ref[0] = acc + part

        @pl.when(b == _B - 1)
        def _write():
            loss_ref[...] = jnp.full((1, 128), acc_ref[0] * ((1.0 + _BETA) / float(_T * _C)), jnp.float32)


_dist = pl.pallas_call(
    _dist_body,
    grid=(_B, _NJ),
    in_specs=[
        pl.BlockSpec((1, _C, _HW), lambda b, j: (b, 0, 0)),
        pl.BlockSpec((_NBLK, _C), lambda b, j: (j, 0)),
        pl.BlockSpec((_NBLK, 1), lambda b, j: (j, 0)),
    ],
    out_specs=[
        pl.BlockSpec((1, 1, _HW), lambda b, j: (b, 0, 0)),
        pl.BlockSpec((1, 128), lambda b, j: (0, 0)),
    ],
    out_shape=[
        jax.ShapeDtypeStruct((_B, 1, _HW), jnp.int32),
        jax.ShapeDtypeStruct((1, 128), jnp.float32),
    ],
    scratch_shapes=[
        pltpu.VMEM((_HW,), jnp.float32),
        pltpu.SMEM((1,), jnp.float32),
    ],
)

_NC, _NS = 2, 16
_NW = _NC * _NS
_BPW = _T // _NW
_CH = 128
_NCH = _BPW // _CH


def _gather_body(tab_ref, idx_ref, out_ref, idx_v, rows_v, sem):
    wid = lax.axis_index("s") * _NC + lax.axis_index("c")
    pltpu.sync_copy(idx_ref.at[pl.ds(wid * _NCH, _NCH)], idx_v)
    for c in range(_NCH):
        pltpu.async_copy(tab_ref.at[idx_v.at[c]], rows_v, sem).wait()
        pltpu.sync_copy(rows_v, out_ref.at[pl.ds(wid * _BPW + c * _CH, _CH)])


@functools.lru_cache(maxsize=1)
def _make_gather():
    return functools.partial(
        pl.kernel,
        out_type=jax.ShapeDtypeStruct((_T, _C), jnp.float32),
        mesh=plsc.VectorSubcoreMesh(core_axis_name="c", subcore_axis_name="s"),
        scratch_types=[
            pltpu.VMEM((_NCH, _CH), jnp.int32),
            pltpu.VMEM((_CH, _C), jnp.float32),
            pltpu.SemaphoreType.DMA,
        ],
    )(_gather_body)


def kernel(z, embedding_weight):
    z_r = z.reshape(_B, _C, _HW)
    w2 = jnp.sum(embedding_weight ** 2, axis=1)
    idx3, loss2 = _dist(z_r, -2.0 * embedding_weight, w2.reshape(_K, 1))
    zq2 = _make_gather()(embedding_weight, idx3.reshape(_NW * _NCH, _CH))
    z_q_out = jnp.moveaxis(zq2.reshape(_B, _H, _W, _C), -1, 1)
    return z_q_out, loss2[0, 0], idx3.reshape(_B, _H, _W)


# speedup vs baseline: 2.8154x; 1.0022x over previous
import functools

import jax
import jax.numpy as jnp
from jax import lax
from jax.experimental import pallas as pl
from jax.experimental.pallas import tpu as pltpu
from jax.experimental.pallas import tpu_sc as plsc

_B, _C, _H, _W = 16, 256, 32, 32
_HW = _H * _W
_T = _B * _HW
_K = 8192
_NBLK = 2048
_NJ = _K // _NBLK
_BETA = 0.25

def _dist_body(x_ref, w_ref, w2_ref, idx_ref, loss_ref, rmin_ref, rarg_ref,
               acc_ref):
    j = pl.program_id(0)
    b = pl.program_id(1)
    x = x_ref[0]
    wblk = w_ref[...]
    mm = lax.dot_general(wblk, x, (((1,), (0,)), ((), ())),
                         preferred_element_type=jnp.float32)
    d = w2_ref[...] + mm
    bmin = jnp.min(d, axis=0)[None, :]
    barg = (jnp.argmin(d, axis=0).astype(jnp.int32) + j * _NBLK)[None, :]
    row = pl.ds(b, 1)

    @pl.when(j == 0)
    def _init():
        rmin_ref[row, :] = bmin
        rarg_ref[row, :] = barg

    @pl.when(j > 0)
    def _update():
        old = rmin_ref[row, :]
        better = bmin < old
        rmin_ref[row, :] = jnp.where(better, bmin, old)
        rarg_ref[row, :] = jnp.where(better, barg, rarg_ref[row, :])

    @pl.when(j == _NJ - 1)
    def _final():
        idx_ref[0, 0, :] = rarg_ref[row, :][0]
        z2 = jnp.sum(x * x, axis=0)
        part = jnp.sum(z2 + rmin_ref[row, :][0])
        acc = jnp.where(b == 0, 0.0, acc_ref[0])
        acc_ref[0] = acc + part

        @pl.when(b == _B - 1)
        def _write():
            loss_ref[...] = jnp.full((1, 128), acc_ref[0] * ((1.0 + _BETA) / float(_T * _C)), jnp.float32)


_dist = pl.pallas_call(
    _dist_body,
    grid=(_NJ, _B),
    in_specs=[
        pl.BlockSpec((1, _C, _HW), lambda j, b: (b, 0, 0)),
        pl.BlockSpec((_NBLK, _C), lambda j, b: (j, 0)),
        pl.BlockSpec((_NBLK, 1), lambda j, b: (j, 0)),
    ],
    out_specs=[
        pl.BlockSpec((1, 1, _HW), lambda j, b: (b, 0, 0)),
        pl.BlockSpec((1, 128), lambda j, b: (0, 0)),
    ],
    out_shape=[
        jax.ShapeDtypeStruct((_B, 1, _HW), jnp.int32),
        jax.ShapeDtypeStruct((1, 128), jnp.float32),
    ],
    scratch_shapes=[
        pltpu.VMEM((_B, _HW), jnp.float32),
        pltpu.VMEM((_B, _HW), jnp.int32),
        pltpu.SMEM((1,), jnp.float32),
    ],
)

_NC, _NS = 2, 16
_NW = _NC * _NS
_BPW = _T // _NW
_CH = 128
_NCH = _BPW // _CH


def _gather_body(tab_ref, idx_ref, out_ref, idx_v, rows_v, sem):
    wid = lax.axis_index("s") * _NC + lax.axis_index("c")
    pltpu.sync_copy(idx_ref.at[pl.ds(wid * _NCH, _NCH)], idx_v)
    for c in range(_NCH):
        pltpu.async_copy(tab_ref.at[idx_v.at[c]], rows_v, sem).wait()
        pltpu.sync_copy(rows_v, out_ref.at[pl.ds(wid * _BPW + c * _CH, _CH)])


@functools.lru_cache(maxsize=1)
def _make_gather():
    return functools.partial(
        pl.kernel,
        out_type=jax.ShapeDtypeStruct((_T, _C), jnp.float32),
        mesh=plsc.VectorSubcoreMesh(core_axis_name="c", subcore_axis_name="s"),
        scratch_types=[
            pltpu.VMEM((_NCH, _CH), jnp.int32),
            pltpu.VMEM((_CH, _C), jnp.float32),
            pltpu.SemaphoreType.DMA,
        ],
    )(_gather_body)


def kernel(z, embedding_weight):
    z_r = z.reshape(_B, _C, _HW)
    w2 = jnp.sum(embedding_weight ** 2, axis=1)
    idx3, loss2 = _dist(z_r, -2.0 * embedding_weight, w2.reshape(_K, 1))
    zq2 = _make_gather()(embedding_weight, idx3.reshape(_NW * _NCH, _CH))
    z_q_out = jnp.moveaxis(zq2.reshape(_B, _H, _W, _C), -1, 1)
    return z_q_out, loss2[0, 0], idx3.reshape(_B, _H, _W)


# speedup vs baseline: 3.0186x; 1.0722x over previous
import functools

import jax
import jax.numpy as jnp
from jax import lax
from jax.experimental import pallas as pl
from jax.experimental.pallas import tpu as pltpu
from jax.experimental.pallas import tpu_sc as plsc

_B, _C, _H, _W = 16, 256, 32, 32
_HW = _H * _W
_T = _B * _HW
_K = 8192
_NBLK = 4096
_NJ = _K // _NBLK
_BETA = 0.25

def _dist_body(x_ref, w_ref, w2_ref, idx_ref, loss_ref, rmin_ref, rarg_ref,
               acc_ref):
    j = pl.program_id(0)
    b = pl.program_id(1)
    x = x_ref[0]
    wblk = w_ref[...]
    mm = lax.dot_general(wblk, x, (((1,), (0,)), ((), ())),
                         preferred_element_type=jnp.float32)
    d = w2_ref[...] + mm
    bmin = jnp.min(d, axis=0)[None, :]
    barg = (jnp.argmin(d, axis=0).astype(jnp.int32) + j * _NBLK)[None, :]
    row = pl.ds(b, 1)

    @pl.when(j == 0)
    def _init():
        rmin_ref[row, :] = bmin
        rarg_ref[row, :] = barg

    @pl.when(j > 0)
    def _update():
        old = rmin_ref[row, :]
        better = bmin < old
        rmin_ref[row, :] = jnp.where(better, bmin, old)
        rarg_ref[row, :] = jnp.where(better, barg, rarg_ref[row, :])

    @pl.when(j == _NJ - 1)
    def _final():
        idx_ref[0, 0, :] = rarg_ref[row, :][0]
        z2 = jnp.sum(x * x, axis=0)
        part = jnp.sum(z2 + rmin_ref[row, :][0])
        acc = jnp.where(b == 0, 0.0, acc_ref[0])
        acc_ref[0] = acc + part

        @pl.when(b == _B - 1)
        def _write():
            loss_ref[...] = jnp.full((1, 128), acc_ref[0] * ((1.0 + _BETA) / float(_T * _C)), jnp.float32)


_dist = pl.pallas_call(
    _dist_body,
    grid=(_NJ, _B),
    in_specs=[
        pl.BlockSpec((1, _C, _HW), lambda j, b: (b, 0, 0)),
        pl.BlockSpec((_NBLK, _C), lambda j, b: (j, 0)),
        pl.BlockSpec((_NBLK, 1), lambda j, b: (j, 0)),
    ],
    out_specs=[
        pl.BlockSpec((1, 1, _HW), lambda j, b: (b, 0, 0)),
        pl.BlockSpec((1, 128), lambda j, b: (0, 0)),
    ],
    out_shape=[
        jax.ShapeDtypeStruct((_B, 1, _HW), jnp.int32),
        jax.ShapeDtypeStruct((1, 128), jnp.float32),
    ],
    scratch_shapes=[
        pltpu.VMEM((_B, _HW), jnp.float32),
        pltpu.VMEM((_B, _HW), jnp.int32),
        pltpu.SMEM((1,), jnp.float32),
    ],
)

_NC, _NS = 2, 16
_NW = _NC * _NS
_BPW = _T // _NW
_CH = 128
_NCH = _BPW // _CH


def _gather_body(tab_ref, idx_ref, out_ref, idx_v, rows_v, sem):
    wid = lax.axis_index("s") * _NC + lax.axis_index("c")
    pltpu.sync_copy(idx_ref.at[pl.ds(wid * _NCH, _NCH)], idx_v)
    for c in range(_NCH):
        pltpu.async_copy(tab_ref.at[idx_v.at[c]], rows_v, sem).wait()
        pltpu.sync_copy(rows_v, out_ref.at[pl.ds(wid * _BPW + c * _CH, _CH)])


@functools.lru_cache(maxsize=1)
def _make_gather():
    return functools.partial(
        pl.kernel,
        out_type=jax.ShapeDtypeStruct((_T, _C), jnp.float32),
        mesh=plsc.VectorSubcoreMesh(core_axis_name="c", subcore_axis_name="s"),
        scratch_types=[
            pltpu.VMEM((_NCH, _CH), jnp.int32),
            pltpu.VMEM((_CH, _C), jnp.float32),
            pltpu.SemaphoreType.DMA,
        ],
    )(_gather_body)


def kernel(z, embedding_weight):
    z_r = z.reshape(_B, _C, _HW)
    w2 = jnp.sum(embedding_weight ** 2, axis=1)
    idx3, loss2 = _dist(z_r, -2.0 * embedding_weight, w2.reshape(_K, 1))
    zq2 = _make_gather()(embedding_weight, idx3.reshape(_NW * _NCH, _CH))
    z_q_out = jnp.moveaxis(zq2.reshape(_B, _H, _W, _C), -1, 1)
    return z_q_out, loss2[0, 0], idx3.reshape(_B, _H, _W)


# speedup vs baseline: 3.1065x; 1.0291x over previous
import functools

import jax
import jax.numpy as jnp
from jax import lax
from jax.experimental import pallas as pl
from jax.experimental.pallas import tpu as pltpu
from jax.experimental.pallas import tpu_sc as plsc

_B, _C, _H, _W = 16, 256, 32, 32
_HW = _H * _W
_T = _B * _HW
_K = 8192
_NBLK = 8192
_NJ = _K // _NBLK
_BETA = 0.25

def _dist_body(x_ref, w_ref, w2_ref, idx_ref, loss_ref, rmin_ref, rarg_ref,
               acc_ref):
    j = pl.program_id(0)
    b = pl.program_id(1)
    x = x_ref[0]
    wblk = w_ref[...]
    mm = lax.dot_general(wblk, x, (((1,), (0,)), ((), ())),
                         preferred_element_type=jnp.float32)
    d = w2_ref[...] + mm
    bmin = jnp.min(d, axis=0)[None, :]
    barg = (jnp.argmin(d, axis=0).astype(jnp.int32) + j * _NBLK)[None, :]
    row = pl.ds(b, 1)

    @pl.when(j == 0)
    def _init():
        rmin_ref[row, :] = bmin
        rarg_ref[row, :] = barg

    @pl.when(j > 0)
    def _update():
        old = rmin_ref[row, :]
        better = bmin < old
        rmin_ref[row, :] = jnp.where(better, bmin, old)
        rarg_ref[row, :] = jnp.where(better, barg, rarg_ref[row, :])

    @pl.when(j == _NJ - 1)
    def _final():
        idx_ref[0, 0, :] = rarg_ref[row, :][0]
        z2 = jnp.sum(x * x, axis=0)
        part = jnp.sum(z2 + rmin_ref[row, :][0])
        acc = jnp.where(b == 0, 0.0, acc_ref[0])
        acc_ref[0] = acc + part

        @pl.when(b == _B - 1)
        def _write():
            loss_ref[...] = jnp.full((1, 128), acc_ref[0] * ((1.0 + _BETA) / float(_T * _C)), jnp.float32)


_dist = pl.pallas_call(
    _dist_body,
    grid=(_NJ, _B),
    in_specs=[
        pl.BlockSpec((1, _C, _HW), lambda j, b: (b, 0, 0)),
        pl.BlockSpec((_NBLK, _C), lambda j, b: (j, 0)),
        pl.BlockSpec((_NBLK, 1), lambda j, b: (j, 0)),
    ],
    out_specs=[
        pl.BlockSpec((1, 1, _HW), lambda j, b: (b, 0, 0)),
        pl.BlockSpec((1, 128), lambda j, b: (0, 0)),
    ],
    out_shape=[
        jax.ShapeDtypeStruct((_B, 1, _HW), jnp.int32),
        jax.ShapeDtypeStruct((1, 128), jnp.float32),
    ],
    scratch_shapes=[
        pltpu.VMEM((_B, _HW), jnp.float32),
        pltpu.VMEM((_B, _HW), jnp.int32),
        pltpu.SMEM((1,), jnp.float32),
    ],
)

_NC, _NS = 2, 16
_NW = _NC * _NS
_BPW = _T // _NW
_CH = 128
_NCH = _BPW // _CH


def _gather_body(tab_ref, idx_ref, out_ref, idx_v, rows_v, sem):
    wid = lax.axis_index("s") * _NC + lax.axis_index("c")
    pltpu.sync_copy(idx_ref.at[pl.ds(wid * _NCH, _NCH)], idx_v)
    for c in range(_NCH):
        pltpu.async_copy(tab_ref.at[idx_v.at[c]], rows_v, sem).wait()
        pltpu.sync_copy(rows_v, out_ref.at[pl.ds(wid * _BPW + c * _CH, _CH)])


@functools.lru_cache(maxsize=1)
def _make_gather():
    return functools.partial(
        pl.kernel,
        out_type=jax.ShapeDtypeStruct((_T, _C), jnp.float32),
        mesh=plsc.VectorSubcoreMesh(core_axis_name="c", subcore_axis_name="s"),
        scratch_types=[
            pltpu.VMEM((_NCH, _CH), jnp.int32),
            pltpu.VMEM((_CH, _C), jnp.float32),
            pltpu.SemaphoreType.DMA,
        ],
    )(_gather_body)


def kernel(z, embedding_weight):
    z_r = z.reshape(_B, _C, _HW)
    w2 = jnp.sum(embedding_weight ** 2, axis=1)
    idx3, loss2 = _dist(z_r, -2.0 * embedding_weight, w2.reshape(_K, 1))
    zq2 = _make_gather()(embedding_weight, idx3.reshape(_NW * _NCH, _CH))
    z_q_out = jnp.moveaxis(zq2.reshape(_B, _H, _W, _C), -1, 1)
    return z_q_out, loss2[0, 0], idx3.reshape(_B, _H, _W)
